# baseline (speedup 1.0000x reference)
import functools

import jax
import jax.numpy as jnp
from jax import lax
from jax.experimental import pallas as pl
from jax.experimental.pallas import tpu as pltpu
from jax.experimental.pallas import tpu_sc as plsc

NENTITY = 1000000
EDIM = 64
NRELATION = 64
BATCH = 16384

NC = 2
NS = 16
NW = NC * NS
ROWS_PER_W = BATCH // NW
CHUNK = 128
NCHUNK = ROWS_PER_W // CHUNK

TC_BLOCK = 2048


def _sc_gather_body(h_hbm, t_hbm, emb_hbm, eh_hbm, et_hbm,
                    hidx_v, tidx_v, bufh, buft, semh, semt):
    wid = lax.axis_index("s") * NC + lax.axis_index("c")
    base = wid * ROWS_PER_W
    pltpu.sync_copy(h_hbm.at[pl.ds(wid * NCHUNK, NCHUNK)], hidx_v)
    pltpu.sync_copy(t_hbm.at[pl.ds(wid * NCHUNK, NCHUNK)], tidx_v)
    waits = []
    for c in range(NCHUNK):
        waits.append(pltpu.async_copy(
            emb_hbm.at[hidx_v.at[c]], bufh.at[pl.ds(c * CHUNK, CHUNK)], semh))
        waits.append(pltpu.async_copy(
            emb_hbm.at[tidx_v.at[c]], buft.at[pl.ds(c * CHUNK, CHUNK)], semt))
    for w in waits:
        w.wait()
    pltpu.sync_copy(bufh, eh_hbm.at[pl.ds(base, ROWS_PER_W)])
    pltpu.sync_copy(buft, et_hbm.at[pl.ds(base, ROWS_PER_W)])


@functools.partial(jax.jit, static_argnames=())
def _sc_gather(h2, t2, emb_e):
    mesh = plsc.VectorSubcoreMesh(core_axis_name="c", subcore_axis_name="s")
    f = pl.kernel(
        _sc_gather_body,
        out_type=[
            jax.ShapeDtypeStruct((BATCH, EDIM), jnp.float32),
            jax.ShapeDtypeStruct((BATCH, EDIM), jnp.float32),
        ],
        mesh=mesh,
        scratch_types=[
            pltpu.VMEM((NCHUNK, CHUNK), jnp.int32),
            pltpu.VMEM((NCHUNK, CHUNK), jnp.int32),
            pltpu.VMEM((ROWS_PER_W, EDIM), jnp.float32),
            pltpu.VMEM((ROWS_PER_W, EDIM), jnp.float32),
            pltpu.SemaphoreType.DMA,
            pltpu.SemaphoreType.DMA,
        ],
        compiler_params=pltpu.CompilerParams(use_tc_tiling_on_sc=False),
    )
    return f(h2, t2, emb_e)


def _tc_body(rel_ref, er_ref, wt_ref, eh_ref, et_ref, out_ref):
    d = eh_ref[...] - et_ref[...]
    y = jnp.dot(d, wt_ref[...], preferred_element_type=jnp.float32)
    rel = rel_ref[...]
    onehot = (rel == lax.broadcasted_iota(jnp.int32, (1, NRELATION), 1)
              ).astype(jnp.float32)
    e_r = jnp.dot(onehot, er_ref[...], preferred_element_type=jnp.float32)
    z = y + e_r
    out_ref[...] = jnp.sum(z * z, axis=1)


def _tc_score(rel2, emb_rel, wt, eh, et):
    grid = (BATCH // TC_BLOCK,)
    return pl.pallas_call(
        _tc_body,
        grid=grid,
        in_specs=[
            pl.BlockSpec((TC_BLOCK, 1), lambda i: (i, 0)),
            pl.BlockSpec((NRELATION, NRELATION), lambda i: (0, 0)),
            pl.BlockSpec((EDIM, EDIM), lambda i: (0, 0)),
            pl.BlockSpec((TC_BLOCK, EDIM), lambda i: (i, 0)),
            pl.BlockSpec((TC_BLOCK, EDIM), lambda i: (i, 0)),
        ],
        out_specs=pl.BlockSpec((TC_BLOCK,), lambda i: (i,)),
        out_shape=jax.ShapeDtypeStruct((BATCH,), jnp.float32),
    )(rel2, emb_rel, wt, eh, et)


def kernel(h, rel, t, emb_e, emb_rel, W):
    h2 = h.reshape(BATCH // CHUNK, CHUNK)
    t2 = t.reshape(BATCH // CHUNK, CHUNK)
    eh, et = _sc_gather(h2, t2, emb_e)
    rel2 = rel.reshape(BATCH, 1)
    wt = W.T
    return _tc_score(rel2, emb_rel, wt, eh, et)
